# Initial kernel scaffold; baseline (speedup 1.0000x reference)
import jax
import jax.numpy as jnp
from jax.experimental import pallas as pl


def kernel(x_user, x_item, ei_ui, ei_iu, Wl0_ui, bl0_ui, Wr0_ui, Wl0_iu, bl0_iu, Wr0_iu, Wl1_ui, bl1_ui, Wr1_ui, Wl1_iu, bl1_iu, Wr1_iu):
    raise NotImplementedError("write your pallas kernel here")


# speedup vs baseline: 5.5082x; 5.5082x over previous
import functools

import jax
import jax.numpy as jnp
from jax import lax
from jax.experimental import pallas as pl
from jax.experimental.pallas import tpu as pltpu
from jax.experimental.pallas import tpu_sc as plsc

N_USER = 10000
N_ITEM = 10000
E = 160000
D = 128

NS = 16
EPT = E // NS
CHUNK = 80
NCH = EPT // CHUNK
RPT = 624
TAIL = N_ITEM - NS * RPT
ZC = 48


def _make_sc_segsum(with_counts):
  outs = [jax.ShapeDtypeStruct((2, N_ITEM, D), jnp.float32)]
  if with_counts:
    outs += [jax.ShapeDtypeStruct((2, N_ITEM, D), jnp.float32)]

  scratch = [pltpu.VMEM((EPT,), jnp.int32),
             pltpu.VMEM((NCH, CHUNK), jnp.int32),
             pltpu.VMEM((CHUNK, D), jnp.float32),
             pltpu.VMEM((CHUNK, D), jnp.float32),
             pltpu.VMEM_SHARED((N_ITEM, D), jnp.float32),
             pltpu.SemaphoreType.DMA,
             pltpu.SemaphoreType.DMA,
             pltpu.SemaphoreType.DMA,
             pltpu.SemaphoreType.DMA]

  mesh = plsc.VectorSubcoreMesh(core_axis_name="c", subcore_axis_name="s")

  def body(tab, srccat, dst4, z128, o128, *rest):
    if with_counts:
      (s_out, c_out, idx2s, idx2d, rA, rB, acc, sGA, sGB, sSA, sSB) = rest
    else:
      (s_out, idx2s, idx2d, rA, rB, acc, sGA, sGB, sSA, sSB) = rest
    cid = lax.axis_index("c")
    sid = lax.axis_index("s")
    row0 = sid * RPT

    def drain(buf, sem):
      pltpu.make_async_copy(z128.at[pl.ds(0, CHUNK)], buf, sem).wait()

    def zero_my_rows():
      pltpu.sync_copy(z128.at[pl.ds(row0, RPT)], acc.at[pl.ds(row0, RPT)])

      @pl.when(sid == NS - 1)
      def _():
        pltpu.sync_copy(z128.at[pl.ds(NS * RPT, TAIL)],
                        acc.at[pl.ds(NS * RPT, TAIL)])

    def copy_out_my_rows(out):
      pltpu.sync_copy(acc.at[pl.ds(row0, RPT)], out.at[cid, pl.ds(row0, RPT)])

      @pl.when(sid == NS - 1)
      def _():
        t0 = NS * RPT
        pltpu.sync_copy(acc.at[pl.ds(t0, TAIL)], out.at[cid, pl.ds(t0, TAIL)])

    pltpu.sync_copy(srccat.at[pl.ds(cid * E + sid * EPT, EPT)], idx2s)
    pltpu.sync_copy(dst4.at[cid, sid], idx2d)

    zero_my_rows()
    plsc.subcore_barrier()

    pltpu.async_copy(tab.at[idx2s.at[pl.ds(0, CHUNK)]], rA, sGA)

    def step(i, carry):
      @pl.when(i % 2 == 0)
      def _():
        @pl.when(i + 1 < NCH)
        def _():
          @pl.when(i > 0)
          def _():
            drain(rB, sSB)
          pltpu.async_copy(tab.at[idx2s.at[pl.ds((i + 1) * CHUNK, CHUNK)]], rB, sGB)
        drain(rA, sGA)
        pltpu.async_copy(rA, acc.at[idx2d.at[i]], sSA, add=True)

      @pl.when(i % 2 == 1)
      def _():
        @pl.when(i + 1 < NCH)
        def _():
          drain(rA, sSA)
          pltpu.async_copy(tab.at[idx2s.at[pl.ds((i + 1) * CHUNK, CHUNK)]], rA, sGA)
        drain(rB, sGB)
        pltpu.async_copy(rB, acc.at[idx2d.at[i]], sSB, add=True)
      return carry
    lax.fori_loop(0, NCH, step, 0)

    drain(rA, sSA)
    drain(rB, sSB)

    plsc.subcore_barrier()
    copy_out_my_rows(s_out)

    if with_counts:
      zero_my_rows()
      plsc.subcore_barrier()
      pltpu.sync_copy(o128, rB)

      def cstep(i, carry):
        @pl.when(i % 2 == 0)
        def _():
          @pl.when(i > 1)
          def _():
            drain(rB, sSA)
          pltpu.async_copy(rB, acc.at[idx2d.at[i]], sSA, add=True)

        @pl.when(i % 2 == 1)
        def _():
          @pl.when(i > 1)
          def _():
            drain(rB, sSB)
          pltpu.async_copy(rB, acc.at[idx2d.at[i]], sSB, add=True)
        return carry
      lax.fori_loop(0, NCH, cstep, 0)

      drain(rB, sSA)
      drain(rB, sSB)

      plsc.subcore_barrier()
      copy_out_my_rows(c_out)

  return pl.kernel(body, out_type=tuple(outs), mesh=mesh,
                   scratch_types=scratch)


_sc_segsum_l0 = _make_sc_segsum(True)
_sc_segsum_l1 = _make_sc_segsum(False)


def _fused_body(relu, s_ref, cnt_ref, x_ref, wl_ref, bl_ref, wr_ref, o_ref):
  inv = 1.0 / jnp.maximum(cnt_ref[:, 0:1], 1.0)
  mean = s_ref[...] * inv
  acc = lax.dot_general(mean, wl_ref[...], (((1,), (0,)), ((), ())),
                        preferred_element_type=jnp.float32,
                        precision=lax.Precision.HIGHEST)
  acc = acc + lax.dot_general(x_ref[...], wr_ref[...], (((1,), (0,)), ((), ())),
                              preferred_element_type=jnp.float32,
                              precision=lax.Precision.HIGHEST)
  acc = acc + bl_ref[...]
  if relu:
    acc = jnp.maximum(acc, 0.0)
  o_ref[...] = acc


def _fused(s, cnt, x, wl, bl, wr, relu):
  n = s.shape[0]
  blk = 1000
  grid = n // blk
  return pl.pallas_call(
      functools.partial(_fused_body, relu),
      grid=(grid,),
      in_specs=[pl.BlockSpec((blk, D), lambda i: (i, 0)),
                pl.BlockSpec((blk, D), lambda i: (i, 0)),
                pl.BlockSpec((blk, D), lambda i: (i, 0)),
                pl.BlockSpec((D, D), lambda i: (0, 0)),
                pl.BlockSpec((1, D), lambda i: (0, 0)),
                pl.BlockSpec((D, D), lambda i: (0, 0))],
      out_specs=pl.BlockSpec((blk, D), lambda i: (i, 0)),
      out_shape=jax.ShapeDtypeStruct((n, D), jnp.float32),
  )(s, cnt, x, wl, bl.reshape(1, D), wr)


def kernel(x_user, x_item, ei_ui, ei_iu,
           Wl0_ui, bl0_ui, Wr0_ui, Wl0_iu, bl0_iu, Wr0_iu,
           Wl1_ui, bl1_ui, Wr1_ui, Wl1_iu, bl1_iu, Wr1_iu):
  z128 = jnp.zeros((N_ITEM, D), jnp.float32)
  o128 = jnp.ones((CHUNK, D), jnp.float32)
  src_ui, dst_ui = ei_ui[0], ei_ui[1]
  src_iu, dst_iu = ei_iu[0], ei_iu[1]

  srccat = jnp.concatenate([src_ui, src_iu + N_ITEM])
  dst4 = jnp.concatenate([dst_ui, dst_iu]).reshape(2, NS, NCH, CHUNK)

  tab0 = jnp.concatenate([x_user, x_item], axis=0)
  s2, c2 = _sc_segsum_l0(tab0, srccat, dst4, z128, o128)
  s_i0, s_u0 = s2[0], s2[1]
  cnt_i, cnt_u = c2[0], c2[1]
  h_item = _fused(s_i0, cnt_i, x_item, Wl0_ui, bl0_ui, Wr0_ui, relu=True)
  h_user = _fused(s_u0, cnt_u, x_user, Wl0_iu, bl0_iu, Wr0_iu, relu=True)

  tab1 = jnp.concatenate([h_user, h_item], axis=0)
  (s2b,) = _sc_segsum_l1(tab1, srccat, dst4, z128, o128)
  s_i1, s_u1 = s2b[0], s2b[1]
  o_item = _fused(s_i1, cnt_i, h_item, Wl1_ui, bl1_ui, Wr1_ui, relu=False)
  o_user = _fused(s_u1, cnt_u, h_user, Wl1_iu, bl1_iu, Wr1_iu, relu=False)
  return (o_user, o_item)


# speedup vs baseline: 5.9043x; 1.0719x over previous
import functools

import jax
import jax.numpy as jnp
from jax import lax
from jax.experimental import pallas as pl
from jax.experimental.pallas import tpu as pltpu
from jax.experimental.pallas import tpu_sc as plsc

N_USER = 10000
N_ITEM = 10000
E = 160000
D = 128

NS = 16
EPT = E // NS
CHUNK = 80
NCH = EPT // CHUNK
RPT = 624
TAIL = N_ITEM - NS * RPT
ZC = 48


def _make_sc_segsum(with_counts):
  outs = [jax.ShapeDtypeStruct((2, N_ITEM, D), jnp.float32)]
  if with_counts:
    outs += [jax.ShapeDtypeStruct((2, N_ITEM, D), jnp.float32)]

  scratch = [pltpu.VMEM((EPT,), jnp.int32),
             pltpu.VMEM((NCH, CHUNK), jnp.int32),
             pltpu.VMEM((CHUNK, D), jnp.float32),
             pltpu.VMEM((CHUNK, D), jnp.float32),
             pltpu.VMEM_SHARED((N_ITEM, D), jnp.float32),
             pltpu.SemaphoreType.DMA,
             pltpu.SemaphoreType.DMA,
             pltpu.SemaphoreType.DMA,
             pltpu.SemaphoreType.DMA]

  mesh = plsc.VectorSubcoreMesh(core_axis_name="c", subcore_axis_name="s")

  def body(tab, srccat, dst4, z128, o128, *rest):
    if with_counts:
      (s_out, c_out, idx2s, idx2d, rA, rB, acc, sGA, sGB, sSA, sSB) = rest
    else:
      (s_out, idx2s, idx2d, rA, rB, acc, sGA, sGB, sSA, sSB) = rest
    cid = lax.axis_index("c")
    sid = lax.axis_index("s")
    row0 = sid * RPT

    def drain(buf, sem):
      pltpu.make_async_copy(z128.at[pl.ds(0, CHUNK)], buf, sem).wait()

    def zero_my_rows():
      pltpu.sync_copy(z128.at[pl.ds(row0, RPT)], acc.at[pl.ds(row0, RPT)])

      @pl.when(sid == NS - 1)
      def _():
        pltpu.sync_copy(z128.at[pl.ds(NS * RPT, TAIL)],
                        acc.at[pl.ds(NS * RPT, TAIL)])

    def copy_out_my_rows(out):
      pltpu.sync_copy(acc.at[pl.ds(row0, RPT)], out.at[cid, pl.ds(row0, RPT)])

      @pl.when(sid == NS - 1)
      def _():
        t0 = NS * RPT
        pltpu.sync_copy(acc.at[pl.ds(t0, TAIL)], out.at[cid, pl.ds(t0, TAIL)])

    pltpu.sync_copy(srccat.at[pl.ds(cid * E + sid * EPT, EPT)], idx2s)
    pltpu.sync_copy(dst4.at[cid, sid], idx2d)

    zero_my_rows()
    plsc.subcore_barrier()

    pltpu.async_copy(tab.at[idx2s.at[pl.ds(0, CHUNK)]], rA, sGA)

    def step(i, carry):
      @pl.when(i % 2 == 0)
      def _():
        @pl.when(i + 1 < NCH)
        def _():
          @pl.when(i > 0)
          def _():
            drain(rB, sSB)
          pltpu.async_copy(tab.at[idx2s.at[pl.ds((i + 1) * CHUNK, CHUNK)]], rB, sGB)
        drain(rA, sGA)
        pltpu.async_copy(rA, acc.at[idx2d.at[i]], sSA, add=True)

      @pl.when(i % 2 == 1)
      def _():
        @pl.when(i + 1 < NCH)
        def _():
          drain(rA, sSA)
          pltpu.async_copy(tab.at[idx2s.at[pl.ds((i + 1) * CHUNK, CHUNK)]], rA, sGA)
        drain(rB, sGB)
        pltpu.async_copy(rB, acc.at[idx2d.at[i]], sSB, add=True)
      return carry
    lax.fori_loop(0, NCH, step, 0)

    drain(rA, sSA)
    drain(rB, sSB)

    plsc.subcore_barrier()
    copy_out_my_rows(s_out)

    if with_counts:
      zero_my_rows()
      plsc.subcore_barrier()
      pltpu.sync_copy(o128, rB)

      def cstep(i, carry):
        @pl.when(i % 2 == 0)
        def _():
          @pl.when(i > 1)
          def _():
            drain(rB, sSA)
          pltpu.async_copy(rB, acc.at[idx2d.at[i]], sSA, add=True)

        @pl.when(i % 2 == 1)
        def _():
          @pl.when(i > 1)
          def _():
            drain(rB, sSB)
          pltpu.async_copy(rB, acc.at[idx2d.at[i]], sSB, add=True)
        return carry
      lax.fori_loop(0, NCH, cstep, 0)

      drain(rB, sSA)
      drain(rB, sSB)

      plsc.subcore_barrier()
      copy_out_my_rows(c_out)

  return pl.kernel(body, out_type=tuple(outs), mesh=mesh,
                   scratch_types=scratch)


_sc_segsum_l0 = _make_sc_segsum(True)
_sc_segsum_l1 = _make_sc_segsum(False)


def _fused_body(relu, s_ref, cnt_ref, x_ref, wl_ref, bl_ref, wr_ref, o_ref):
  inv = 1.0 / jnp.maximum(cnt_ref[:, 0:1], 1.0)
  mean = s_ref[...] * inv
  acc = lax.dot_general(mean, wl_ref[...], (((1,), (0,)), ((), ())),
                        preferred_element_type=jnp.float32)
  acc = acc + lax.dot_general(x_ref[...], wr_ref[...], (((1,), (0,)), ((), ())),
                              preferred_element_type=jnp.float32)
  acc = acc + bl_ref[...]
  if relu:
    acc = jnp.maximum(acc, 0.0)
  o_ref[...] = acc


def _fused(s, cnt, x, wl, bl, wr, relu):
  n = s.shape[0]
  blk = 1000
  grid = n // blk
  return pl.pallas_call(
      functools.partial(_fused_body, relu),
      grid=(grid,),
      in_specs=[pl.BlockSpec((blk, D), lambda i: (i, 0)),
                pl.BlockSpec((blk, D), lambda i: (i, 0)),
                pl.BlockSpec((blk, D), lambda i: (i, 0)),
                pl.BlockSpec((D, D), lambda i: (0, 0)),
                pl.BlockSpec((1, D), lambda i: (0, 0)),
                pl.BlockSpec((D, D), lambda i: (0, 0))],
      out_specs=pl.BlockSpec((blk, D), lambda i: (i, 0)),
      out_shape=jax.ShapeDtypeStruct((n, D), jnp.float32),
  )(s, cnt, x, wl, bl.reshape(1, D), wr)


def kernel(x_user, x_item, ei_ui, ei_iu,
           Wl0_ui, bl0_ui, Wr0_ui, Wl0_iu, bl0_iu, Wr0_iu,
           Wl1_ui, bl1_ui, Wr1_ui, Wl1_iu, bl1_iu, Wr1_iu):
  z128 = jnp.zeros((N_ITEM, D), jnp.float32)
  o128 = jnp.ones((CHUNK, D), jnp.float32)
  src_ui, dst_ui = ei_ui[0], ei_ui[1]
  src_iu, dst_iu = ei_iu[0], ei_iu[1]

  srccat = jnp.concatenate([src_ui, src_iu + N_ITEM])
  dst4 = jnp.concatenate([dst_ui, dst_iu]).reshape(2, NS, NCH, CHUNK)

  tab0 = jnp.concatenate([x_user, x_item], axis=0)
  s2, c2 = _sc_segsum_l0(tab0, srccat, dst4, z128, o128)
  s_i0, s_u0 = s2[0], s2[1]
  cnt_i, cnt_u = c2[0], c2[1]
  h_item = _fused(s_i0, cnt_i, x_item, Wl0_ui, bl0_ui, Wr0_ui, relu=True)
  h_user = _fused(s_u0, cnt_u, x_user, Wl0_iu, bl0_iu, Wr0_iu, relu=True)

  tab1 = jnp.concatenate([h_user, h_item], axis=0)
  (s2b,) = _sc_segsum_l1(tab1, srccat, dst4, z128, o128)
  s_i1, s_u1 = s2b[0], s2b[1]
  o_item = _fused(s_i1, cnt_i, h_item, Wl1_ui, bl1_ui, Wr1_ui, relu=False)
  o_user = _fused(s_u1, cnt_u, h_user, Wl1_iu, bl1_iu, Wr1_iu, relu=False)
  return (o_user, o_item)
